# baseline (speedup 1.0000x reference)
import functools

import jax
import jax.numpy as jnp
from jax import lax
from jax.experimental import pallas as pl
from jax.experimental.pallas import tpu as pltpu
from jax.experimental.pallas import tpu_sc as plsc

_LANES = 16
_CORES = 2
_SUBCORES = 16
_TILES = _CORES * _SUBCORES
_GCHUNK = 128


@functools.partial(jax.jit, static_argnames=("n_g",))
def _sc_partials(table, idx, rew, *, n_g):
    mesh = plsc.VectorSubcoreMesh(
        core_axis_name="c", subcore_axis_name="s",
        num_cores=_CORES, num_subcores=_SUBCORES,
    )

    @functools.partial(
        pl.kernel,
        mesh=mesh,
        out_type=jax.ShapeDtypeStruct((_TILES, _LANES), jnp.float32),
        scratch_types=[
            pltpu.VMEM((n_g, _GCHUNK), jnp.int32),
            pltpu.VMEM((n_g, _GCHUNK), jnp.float32),
            pltpu.VMEM((n_g, _GCHUNK), jnp.float32),
            pltpu.VMEM((_LANES,), jnp.float32),
            pltpu.SemaphoreType.DMA,
        ],
    )
    def body(table_hbm, idx_hbm, rew_hbm, out_hbm,
             idx_v, rew_v, vals_v, part_v, sem):
        wid = lax.axis_index("s") * _CORES + lax.axis_index("c")

        pltpu.sync_copy(idx_hbm.at[wid], idx_v)
        pltpu.sync_copy(rew_hbm.at[wid], rew_v)

        descs = [
            pltpu.make_async_copy(table_hbm.at[idx_v.at[g]], vals_v.at[g], sem)
            for g in range(n_g)
        ]
        for d in descs:
            d.start()
        for d in descs:
            d.wait()

        acc = jnp.zeros((_LANES,), jnp.float32)
        for g in range(n_g):
            for k in range(_GCHUNK // _LANES):
                v = vals_v[g, pl.ds(k * _LANES, _LANES)]
                w = rew_v[g, pl.ds(k * _LANES, _LANES)]
                acc = acc + v * w
        part_v[...] = acc
        pltpu.sync_copy(part_v, out_hbm.at[wid])

    return body(table, idx, rew)


def _reduce_body(p_ref, o_ref):
    o_ref[0, 0] = -jnp.sum(p_ref[...])


@jax.jit
def _tc_reduce(partials):
    out = pl.pallas_call(
        _reduce_body,
        out_shape=jax.ShapeDtypeStruct((1, 1), jnp.float32),
        out_specs=pl.BlockSpec(memory_space=pltpu.SMEM),
    )(partials)
    return out[0, 0]


def kernel(seqLogprobs, reward, batchsize_cap, target):
    b, t, vocab = seqLogprobs.shape
    rows = b * t
    n_g = -(-rows // (_TILES * _GCHUNK))
    padded = _TILES * n_g * _GCHUNK

    table = jnp.transpose(seqLogprobs, (1, 2, 0)).reshape(-1)
    r = jnp.arange(rows, dtype=jnp.int32)
    flat_idx = ((r % t) * (vocab * b)
                + target.astype(jnp.int32) * b
                + r // t)
    idx = jnp.pad(flat_idx, (0, padded - rows)).reshape(_TILES, n_g, _GCHUNK)
    rew = jnp.pad(reward.astype(jnp.float32),
                  (0, padded - rows)).reshape(_TILES, n_g, _GCHUNK)

    partials = _sc_partials(table, idx, rew, n_g=n_g)
    return _tc_reduce(partials)


# speedup vs baseline: 1.0467x; 1.0467x over previous
import functools

import jax
import jax.numpy as jnp
from jax import lax
from jax.experimental import pallas as pl
from jax.experimental.pallas import tpu as pltpu
from jax.experimental.pallas import tpu_sc as plsc

_LANES = 16
_CORES = 1
_SUBCORES = 16
_TILES = _CORES * _SUBCORES
_GCHUNK = 128


@functools.partial(jax.jit, static_argnames=("n_g",))
def _sc_partials(table, idx, rew, *, n_g):
    mesh = plsc.VectorSubcoreMesh(
        core_axis_name="c", subcore_axis_name="s",
        num_cores=_CORES, num_subcores=_SUBCORES,
    )

    @functools.partial(
        pl.kernel,
        mesh=mesh,
        out_type=jax.ShapeDtypeStruct((_TILES, _LANES), jnp.float32),
        scratch_types=[
            pltpu.VMEM((n_g, _GCHUNK), jnp.int32),
            pltpu.VMEM((n_g, _GCHUNK), jnp.float32),
            pltpu.VMEM((n_g, _GCHUNK), jnp.float32),
            pltpu.VMEM((_LANES,), jnp.float32),
            pltpu.SemaphoreType.DMA,
        ],
    )
    def body(table_hbm, idx_hbm, rew_hbm, out_hbm,
             idx_v, rew_v, vals_v, part_v, sem):
        wid = lax.axis_index("s") * _CORES + lax.axis_index("c")

        pltpu.sync_copy(idx_hbm.at[wid], idx_v)
        pltpu.sync_copy(rew_hbm.at[wid], rew_v)

        descs = [
            pltpu.make_async_copy(table_hbm.at[idx_v.at[g]], vals_v.at[g], sem)
            for g in range(n_g)
        ]
        for d in descs:
            d.start()
        acc = jnp.zeros((_LANES,), jnp.float32)
        for g in range(n_g):
            descs[g].wait()
            for k in range(_GCHUNK // _LANES):
                v = vals_v[g, pl.ds(k * _LANES, _LANES)]
                w = rew_v[g, pl.ds(k * _LANES, _LANES)]
                acc = acc + v * w
        part_v[...] = acc
        pltpu.sync_copy(part_v, out_hbm.at[wid])

    return body(table, idx, rew)


def _reduce_body(p_ref, o_ref):
    o_ref[0, 0] = -jnp.sum(p_ref[...])


@jax.jit
def _tc_reduce(partials):
    out = pl.pallas_call(
        _reduce_body,
        out_shape=jax.ShapeDtypeStruct((1, 1), jnp.float32),
        out_specs=pl.BlockSpec(memory_space=pltpu.SMEM),
    )(partials)
    return out[0, 0]


def kernel(seqLogprobs, reward, batchsize_cap, target):
    b, t, vocab = seqLogprobs.shape
    rows = b * t
    n_g = -(-rows // (_TILES * _GCHUNK))
    padded = _TILES * n_g * _GCHUNK

    table = jnp.transpose(seqLogprobs, (1, 2, 0)).reshape(-1)
    r = jnp.arange(rows, dtype=jnp.int32)
    flat_idx = ((r % t) * (vocab * b)
                + target.astype(jnp.int32) * b
                + r // t)
    idx = jnp.pad(flat_idx, (0, padded - rows)).reshape(_TILES, n_g, _GCHUNK)
    rew = jnp.pad(reward.astype(jnp.float32),
                  (0, padded - rows)).reshape(_TILES, n_g, _GCHUNK)

    partials = _sc_partials(table, idx, rew, n_g=n_g)
    return _tc_reduce(partials)


# speedup vs baseline: 1.0484x; 1.0016x over previous
import functools

import jax
import jax.numpy as jnp
from jax import lax
from jax.experimental import pallas as pl
from jax.experimental.pallas import tpu as pltpu
from jax.experimental.pallas import tpu_sc as plsc

_LANES = 16
_CORES = 1
_SUBCORES = 16
_TILES = _CORES * _SUBCORES
_GCHUNK = 128


@functools.partial(jax.jit, static_argnames=("n_g",))
def _sc_partials(table, idx, rew, *, n_g):
    per_tile = n_g * _GCHUNK
    mesh = plsc.VectorSubcoreMesh(
        core_axis_name="c", subcore_axis_name="s",
        num_cores=_CORES, num_subcores=_SUBCORES,
    )

    @functools.partial(
        pl.kernel,
        mesh=mesh,
        out_type=jax.ShapeDtypeStruct((_TILES, _LANES), jnp.float32),
        scratch_types=[
            pltpu.VMEM((per_tile,), jnp.int32),
            pltpu.VMEM((per_tile,), jnp.float32),
            pltpu.VMEM((n_g, _GCHUNK), jnp.float32),
            pltpu.VMEM((_LANES,), jnp.float32),
            pltpu.SemaphoreType.DMA,
        ],
    )
    def body(table_hbm, idx_hbm, rew_hbm, out_hbm,
             idx_v, rew_v, vals_v, part_v, sem):
        wid = lax.axis_index("s") * _CORES + lax.axis_index("c")
        base = wid * per_tile

        pltpu.sync_copy(idx_hbm.at[pl.ds(base, per_tile)], idx_v)
        pltpu.sync_copy(rew_hbm.at[pl.ds(base, per_tile)], rew_v)

        descs = [
            pltpu.make_async_copy(
                table_hbm.at[idx_v.at[pl.ds(g * _GCHUNK, _GCHUNK)]],
                vals_v.at[g], sem)
            for g in range(n_g)
        ]
        for d in descs:
            d.start()
        acc = jnp.zeros((_LANES,), jnp.float32)
        for g in range(n_g):
            descs[g].wait()
            for k in range(_GCHUNK // _LANES):
                v = vals_v[g, pl.ds(k * _LANES, _LANES)]
                w = rew_v[pl.ds(g * _GCHUNK + k * _LANES, _LANES)]
                acc = acc + v * w
        part_v[...] = acc
        pltpu.sync_copy(part_v, out_hbm.at[wid])

    return body(table, idx, rew)


def _reduce_body(p_ref, o_ref):
    o_ref[0, 0] = -jnp.sum(p_ref[...])


@jax.jit
def _tc_reduce(partials):
    out = pl.pallas_call(
        _reduce_body,
        out_shape=jax.ShapeDtypeStruct((1, 1), jnp.float32),
        out_specs=pl.BlockSpec(memory_space=pltpu.SMEM),
    )(partials)
    return out[0, 0]


def kernel(seqLogprobs, reward, batchsize_cap, target):
    b, t, vocab = seqLogprobs.shape
    rows = b * t
    n_g = -(-rows // (_TILES * _GCHUNK))
    padded = _TILES * n_g * _GCHUNK

    table = jnp.transpose(seqLogprobs, (1, 2, 0)).reshape(-1)
    r = jnp.arange(rows, dtype=jnp.int32)
    flat_idx = ((r % t) * (vocab * b)
                + target.astype(jnp.int32) * b
                + r // t)
    idx = jnp.pad(flat_idx, (0, padded - rows))
    rew = jnp.pad(reward.astype(jnp.float32), (0, padded - rows))

    partials = _sc_partials(table, idx, rew, n_g=n_g)
    return _tc_reduce(partials)
